# baseline (speedup 1.0000x reference)
import functools
import jax
import jax.numpy as jnp
from jax import lax
from jax.experimental import pallas as pl
from jax.experimental.pallas import tpu as pltpu
from jax.experimental.pallas import tpu_sc as plsc

N = 10000
H = 128
NUM_LAYERS = 3
RES_FREQ = 2

NW = 32
CH = 128
E2 = 640000
NCH = 160
EPW = NCH * CH
EP = NW * EPW
IXR = EP // CH + 8
PADI = IXR * CH - E2
ZR = 10240
DUMMY = N
RPT = ZR // 16
ORB = 624
CW = 128

_SC_MESH = plsc.VectorSubcoreMesh(core_axis_name="c", subcore_axis_name="s")


def _zero_rows(ref, nrows, width):
    z = jnp.zeros((16,), jnp.float32)

    def body(i, _):
        for j in range(width // 16):
            ref[i, pl.ds(j * 16, 16)] = z
        return 0

    lax.fori_loop(0, nrows, body, 0)


def _staged_readout(acc, stage, out_hbm, c, s):
    for t in range(5):
        base = s * ORB + t * CH
        pltpu.sync_copy(acc.at[pl.ds(base, CH)], stage)
        pltpu.sync_copy(stage, out_hbm.at[c, pl.ds(base, CH)])


@functools.partial(
    pl.kernel,
    mesh=_SC_MESH,
    out_type=jax.ShapeDtypeStruct((2, N, H), jnp.float32),
    scratch_types=(
        [pltpu.VMEM((CH,), jnp.int32)] * 8
        + [pltpu.VMEM((CH, H), jnp.float32)] * 2
        + [pltpu.VMEM_SHARED((ZR, H), jnp.float32)]
        + [pltpu.SemaphoreType.DMA] * 6
    ),
)
def _sc_msg(y_hbm, src_hbm, dst_hbm, z_hbm,
            si0, si1, si2, si3, di0, di1, di2, di3, rows0, rows1, zsh,
            sg0, sg1, sm0, sm1, sm2, sm3):
    c = lax.axis_index("c")
    s = lax.axis_index("s")
    wid = s * 2 + c
    base = wid * NCH
    sidx = [si0, si1, si2, si3]
    didx = [di0, di1, di2, di3]
    rows = [rows0, rows1]
    sg = [sg0, sg1]
    sm = [sm0, sm1, sm2, sm3]

    _zero_rows(rows0, CH, H)
    for t in range(RPT // CH):
        pltpu.sync_copy(rows0, zsh.at[pl.ds(s * RPT + t * CH, CH)])
    plsc.subcore_barrier()

    def idx_load(i, q, copy):
        off = (base + i) * CH
        copy(src_hbm.at[pl.ds(off, CH)], sidx[q], sm[q])
        copy(dst_hbm.at[pl.ds(off, CH)], didx[q], sm[q])

    def idx_wait(q):
        off = base * CH
        pltpu.make_async_copy(src_hbm.at[pl.ds(off, CH)], sidx[q],
                              sm[q]).wait()
        pltpu.make_async_copy(dst_hbm.at[pl.ds(off, CH)], didx[q],
                              sm[q]).wait()

    idx_load(0, 0, lambda a, b, _: pltpu.sync_copy(a, b))
    idx_load(1, 1, lambda a, b, _: pltpu.sync_copy(a, b))
    idx_load(2, 2, pltpu.async_copy)
    idx_load(3, 3, pltpu.async_copy)
    pltpu.async_copy(y_hbm.at[si0], rows0, sg0)
    pltpu.async_copy(y_hbm.at[si1], rows1, sg1)

    def quad(g, _):
        i0 = g * 4
        for q in range(4):
            i = i0 + q
            r = q % 2
            q2 = (q + 2) % 4
            pltpu.make_async_copy(y_hbm.at[sidx[q]], rows[r], sg[r]).wait()
            pltpu.sync_copy(rows[r], zsh.at[didx[q]], add=True)
            idx_load(i + 4, q, pltpu.async_copy)
            idx_wait(q2)
            pltpu.async_copy(y_hbm.at[sidx[q2]], rows[r], sg[r])
        return 0

    lax.fori_loop(0, NCH // 4, quad, 0)
    idx_wait(2)
    idx_wait(3)
    pltpu.make_async_copy(y_hbm.at[si0], rows0, sg0).wait()
    pltpu.make_async_copy(y_hbm.at[si1], rows1, sg1).wait()
    plsc.subcore_barrier()
    _staged_readout(zsh, rows0, z_hbm, c, s)


@functools.partial(
    pl.kernel,
    mesh=_SC_MESH,
    out_type=jax.ShapeDtypeStruct((2, N, CW), jnp.float32),
    scratch_types=[
        pltpu.VMEM((CH,), jnp.int32),
        pltpu.VMEM((CH,), jnp.int32),
        pltpu.VMEM((CH, CW), jnp.float32),
        pltpu.VMEM_SHARED((ZR, CW), jnp.float32),
        pltpu.SemaphoreType.DMA,
        pltpu.SemaphoreType.DMA,
    ],
)
def _sc_cnt(idx_hbm, c_hbm, ci0, ci1, obuf, csh, sm0, sm1):
    c = lax.axis_index("c")
    s = lax.axis_index("s")
    wid = s * 2 + c
    base = wid * NCH
    cidx = [ci0, ci1]
    sm = [sm0, sm1]

    _zero_rows(obuf, CH, CW)
    for t in range(RPT // CH):
        pltpu.sync_copy(obuf, csh.at[pl.ds(s * RPT + t * CH, CH)])
    plsc.subcore_barrier()

    one = jnp.ones((16,), jnp.float32)

    def fill(i, _):
        for j in range(CW // 16):
            obuf[i, pl.ds(j * 16, 16)] = one
        return 0

    lax.fori_loop(0, CH, fill, 0)

    def idx_load(i, q):
        pltpu.async_copy(idx_hbm.at[pl.ds((base + i) * CH, CH)], cidx[q],
                         sm[q])

    def idx_wait(q):
        pltpu.make_async_copy(idx_hbm.at[pl.ds(base * CH, CH)], cidx[q],
                              sm[q]).wait()

    idx_load(0, 0)
    idx_load(1, 1)

    def body(g, _):
        i0 = g * 2
        for q in range(2):
            idx_wait(q)
            pltpu.sync_copy(obuf, csh.at[cidx[q]], add=True)
            idx_load(i0 + q + 2, q)
        return 0

    lax.fori_loop(0, NCH // 2, body, 0)
    idx_wait(0)
    idx_wait(1)
    plsc.subcore_barrier()
    _staged_readout(csh, obuf, c_hbm, c, s)


def _relu(v):
    return jnp.maximum(v, 0.0)


def _mm(a, b):
    return jnp.dot(a, b, preferred_element_type=jnp.float32)


def _tc_in_body(x, w1, b1, w2, b2, wg, cp, wr, oh, oy, od):
    h = _mm(_relu(_mm(x[...], w1[...]) + b1[...]), w2[...]) + b2[...]
    cnt = (cp[0] + cp[1])[:, 0:1]
    dinv = lax.rsqrt(wr[0, 0] * cnt + 2.0)
    oh[...] = h
    od[...] = dinv
    oy[...] = dinv * _mm(h, wg[...])


def _tc_in(x, w1, b1, w2, b2, wg, cp, wr):
    return pl.pallas_call(
        _tc_in_body,
        out_shape=(
            jax.ShapeDtypeStruct((N, H), jnp.float32),
            jax.ShapeDtypeStruct((N, H), jnp.float32),
            jax.ShapeDtypeStruct((N, 1), jnp.float32),
        ),
    )(x, w1, b1.reshape(1, H), w2, b2.reshape(1, H), wg, cp, wr)


def _tc_post_body(has_skip, is_final, *refs):
    if is_final:
        (zp, y, h, dinv, wr, gb, ms, nw, nb, skw, skb,
         hw1, hb1, hw2, hb2, hw3, hb3, out) = refs
    elif has_skip:
        (zp, y, h, dinv, wr, gb, ms, nw, nb, skw, skb, wgn, oh, oy) = refs
    else:
        (zp, y, h, dinv, wr, gb, ms, nw, nb, wgn, oh, oy) = refs
    di = dinv[...]
    g = di * (wr[0, 0] * (zp[0] + zp[1]) + 2.0 * y[...]) + gb[...]
    mean = jnp.mean(g, axis=0, keepdims=True)
    o = g - ms[...] * mean
    var = jnp.mean(o * o, axis=0, keepdims=True)
    g = nw[...] * o * lax.rsqrt(var + 1e-5) + nb[...]
    g = jnp.where(g > 0, g, jnp.exp(g) - 1.0)
    if has_skip:
        g = g + _mm(h[...], skw[...]) + skb[...]
    if is_final:
        q = _relu(_mm(g, hw1[...]) + hb1[...])
        q = _relu(_mm(q, hw2[...]) + hb2[...])
        q = _mm(q, hw3[...]) + hb3[...]
        out[...] = 1.0 / (1.0 + jnp.exp(-q))
    else:
        oh[...] = g
        oy[...] = di * _mm(g, wgn[...])


def _tc_mid(has_skip, args):
    return pl.pallas_call(
        functools.partial(_tc_post_body, has_skip, False),
        out_shape=(
            jax.ShapeDtypeStruct((N, H), jnp.float32),
            jax.ShapeDtypeStruct((N, H), jnp.float32),
        ),
    )(*args)


def _tc_final(args):
    return pl.pallas_call(
        functools.partial(_tc_post_body, True, True),
        out_shape=jax.ShapeDtypeStruct((N, 1), jnp.float32),
    )(*args)


def kernel(x, edge_index, batch, params):
    p = params
    wr = p['edge_weight'].reshape(1, 1)
    row, col = edge_index[0], edge_index[1]
    pad0 = jnp.zeros((PADI,), jnp.int32)
    padd = jnp.full((PADI,), DUMMY, jnp.int32)
    srcg = jnp.concatenate([row, col, pad0])
    cidx = jnp.concatenate([row, col, padd])
    dst = jnp.concatenate([col, row, padd])

    cp = _sc_cnt(cidx)
    h, y, dinv = _tc_in(x, p['in_W1'], p['in_b1'], p['in_W2'], p['in_b2'],
                        p['gcn_W0'], cp, wr)

    for i in range(NUM_LAYERS):
        zp = _sc_msg(y, srcg, dst)
        gb = p['gcn_b%d' % i].reshape(1, H)
        ms = p['norm_ms%d' % i].reshape(1, H)
        nw = p['norm_w%d' % i].reshape(1, H)
        nb = p['norm_b%d' % i].reshape(1, H)
        if i < NUM_LAYERS - 1:
            args = [zp, y, h, dinv, wr, gb, ms, nw, nb]
            has_skip = i % RES_FREQ == 0
            if has_skip:
                args += [p['skip_W%d' % i], p['skip_b%d' % i].reshape(1, H)]
            args += [p['gcn_W%d' % (i + 1)]]
            h, y = _tc_mid(has_skip, args)
        else:
            args = [zp, y, h, dinv, wr, gb, ms, nw, nb,
                    p['skip_W%d' % i], p['skip_b%d' % i].reshape(1, H),
                    p['head_W1'], p['head_b1'].reshape(1, H),
                    p['head_W2'], p['head_b2'].reshape(1, H // 2),
                    p['head_W3'], p['head_b3'].reshape(1, 1)]
            out = _tc_final(args)
    return out.reshape(-1)


# speedup vs baseline: 1.0010x; 1.0010x over previous
import functools
import jax
import jax.numpy as jnp
from jax import lax
from jax.experimental import pallas as pl
from jax.experimental.pallas import tpu as pltpu
from jax.experimental.pallas import tpu_sc as plsc

N = 10000
H = 128
NUM_LAYERS = 3
RES_FREQ = 2

NW = 32
CH = 128
E2 = 640000
NCH = 160
EPW = NCH * CH
EP = NW * EPW
IXR = EP // CH + 8
PADI = IXR * CH - E2
ZR = 10240
DUMMY = N
RPT = ZR // 16
ORB = 624
CW = 128

_SC_MESH = plsc.VectorSubcoreMesh(core_axis_name="c", subcore_axis_name="s")


def _zero_rows(ref, nrows, width):
    z = jnp.zeros((16,), jnp.float32)

    def body(i, _):
        for j in range(width // 16):
            ref[i, pl.ds(j * 16, 16)] = z
        return 0

    lax.fori_loop(0, nrows, body, 0)


def _staged_readout(acc, stage, out_hbm, c, s):
    for t in range(5):
        base = s * ORB + t * CH
        pltpu.sync_copy(acc.at[pl.ds(base, CH)], stage)
        pltpu.sync_copy(stage, out_hbm.at[c, pl.ds(base, CH)])


@functools.partial(
    pl.kernel,
    mesh=_SC_MESH,
    out_type=jax.ShapeDtypeStruct((2, N, H), jnp.float32),
    scratch_types=[
        pltpu.VMEM((CH,), jnp.int32),
        pltpu.VMEM((CH,), jnp.int32),
        pltpu.VMEM((CH,), jnp.int32),
        pltpu.VMEM((CH,), jnp.int32),
        pltpu.VMEM((CH, H), jnp.float32),
        pltpu.VMEM((CH, H), jnp.float32),
        pltpu.VMEM_SHARED((ZR, H), jnp.float32),
        pltpu.SemaphoreType.DMA,
        pltpu.SemaphoreType.DMA,
    ],
)
def _sc_msg(y_hbm, src_hbm, dst_hbm, z_hbm, sidx0, didx0, sidx1, didx1,
            rows0, rows1, zsh, sem0, sem1):
    c = lax.axis_index("c")
    s = lax.axis_index("s")
    wid = s * 2 + c
    base = wid * EPW

    _zero_rows(rows0, CH, H)
    for t in range(RPT // CH):
        pltpu.sync_copy(rows0, zsh.at[pl.ds(s * RPT + t * CH, CH)])
    plsc.subcore_barrier()

    def load_idx(off, sidx, didx):
        pltpu.sync_copy(src_hbm.at[pl.ds(off, CH)], sidx)
        pltpu.sync_copy(dst_hbm.at[pl.ds(off, CH)], didx)

    load_idx(base, sidx0, didx0)
    pltpu.async_copy(y_hbm.at[sidx0], rows0, sem0)

    def pair(j, _):
        i = j * 2
        load_idx(base + (i + 1) * CH, sidx1, didx1)
        pltpu.async_copy(y_hbm.at[sidx1], rows1, sem1)
        pltpu.make_async_copy(y_hbm.at[sidx0], rows0, sem0).wait()
        pltpu.sync_copy(rows0, zsh.at[didx0], add=True)
        load_idx(base + (i + 2) * CH, sidx0, didx0)
        pltpu.async_copy(y_hbm.at[sidx0], rows0, sem0)
        pltpu.make_async_copy(y_hbm.at[sidx1], rows1, sem1).wait()
        pltpu.sync_copy(rows1, zsh.at[didx1], add=True)
        return 0

    lax.fori_loop(0, NCH // 2, pair, 0)
    pltpu.make_async_copy(y_hbm.at[sidx0], rows0, sem0).wait()
    plsc.subcore_barrier()
    _staged_readout(zsh, rows0, z_hbm, c, s)


@functools.partial(
    pl.kernel,
    mesh=_SC_MESH,
    out_type=jax.ShapeDtypeStruct((2, N, CW), jnp.float32),
    scratch_types=[
        pltpu.VMEM((CH,), jnp.int32),
        pltpu.VMEM((CH,), jnp.int32),
        pltpu.VMEM((CH, CW), jnp.float32),
        pltpu.VMEM_SHARED((ZR, CW), jnp.float32),
        pltpu.SemaphoreType.DMA,
        pltpu.SemaphoreType.DMA,
    ],
)
def _sc_cnt(idx_hbm, c_hbm, ci0, ci1, obuf, csh, sm0, sm1):
    c = lax.axis_index("c")
    s = lax.axis_index("s")
    wid = s * 2 + c
    base = wid * NCH
    cidx = [ci0, ci1]
    sm = [sm0, sm1]

    _zero_rows(obuf, CH, CW)
    for t in range(RPT // CH):
        pltpu.sync_copy(obuf, csh.at[pl.ds(s * RPT + t * CH, CH)])
    plsc.subcore_barrier()

    one = jnp.ones((16,), jnp.float32)

    def fill(i, _):
        for j in range(CW // 16):
            obuf[i, pl.ds(j * 16, 16)] = one
        return 0

    lax.fori_loop(0, CH, fill, 0)

    def idx_load(i, q):
        pltpu.async_copy(idx_hbm.at[pl.ds((base + i) * CH, CH)], cidx[q],
                         sm[q])

    def idx_wait(q):
        pltpu.make_async_copy(idx_hbm.at[pl.ds(base * CH, CH)], cidx[q],
                              sm[q]).wait()

    idx_load(0, 0)
    idx_load(1, 1)

    def body(g, _):
        i0 = g * 2
        for q in range(2):
            idx_wait(q)
            pltpu.sync_copy(obuf, csh.at[cidx[q]], add=True)
            idx_load(i0 + q + 2, q)
        return 0

    lax.fori_loop(0, NCH // 2, body, 0)
    idx_wait(0)
    idx_wait(1)
    plsc.subcore_barrier()
    _staged_readout(csh, obuf, c_hbm, c, s)


def _relu(v):
    return jnp.maximum(v, 0.0)


def _mm(a, b):
    return jnp.dot(a, b, preferred_element_type=jnp.float32)


def _tc_in_body(x, w1, b1, w2, b2, wg, cp, wr, oh, oy, od):
    h = _mm(_relu(_mm(x[...], w1[...]) + b1[...]), w2[...]) + b2[...]
    cnt = (cp[0] + cp[1])[:, 0:1]
    dinv = lax.rsqrt(wr[0, 0] * cnt + 2.0)
    oh[...] = h
    od[...] = dinv
    oy[...] = dinv * _mm(h, wg[...])


def _tc_in(x, w1, b1, w2, b2, wg, cp, wr):
    return pl.pallas_call(
        _tc_in_body,
        out_shape=(
            jax.ShapeDtypeStruct((N, H), jnp.float32),
            jax.ShapeDtypeStruct((N, H), jnp.float32),
            jax.ShapeDtypeStruct((N, 1), jnp.float32),
        ),
    )(x, w1, b1.reshape(1, H), w2, b2.reshape(1, H), wg, cp, wr)


def _tc_post_body(has_skip, is_final, *refs):
    if is_final:
        (zp, y, h, dinv, wr, gb, ms, nw, nb, skw, skb,
         hw1, hb1, hw2, hb2, hw3, hb3, out) = refs
    elif has_skip:
        (zp, y, h, dinv, wr, gb, ms, nw, nb, skw, skb, wgn, oh, oy) = refs
    else:
        (zp, y, h, dinv, wr, gb, ms, nw, nb, wgn, oh, oy) = refs
    di = dinv[...]
    g = di * (wr[0, 0] * (zp[0] + zp[1]) + 2.0 * y[...]) + gb[...]
    mean = jnp.mean(g, axis=0, keepdims=True)
    o = g - ms[...] * mean
    var = jnp.mean(o * o, axis=0, keepdims=True)
    g = nw[...] * o * lax.rsqrt(var + 1e-5) + nb[...]
    g = jnp.where(g > 0, g, jnp.exp(g) - 1.0)
    if has_skip:
        g = g + _mm(h[...], skw[...]) + skb[...]
    if is_final:
        q = _relu(_mm(g, hw1[...]) + hb1[...])
        q = _relu(_mm(q, hw2[...]) + hb2[...])
        q = _mm(q, hw3[...]) + hb3[...]
        out[...] = 1.0 / (1.0 + jnp.exp(-q))
    else:
        oh[...] = g
        oy[...] = di * _mm(g, wgn[...])


def _tc_mid(has_skip, args):
    return pl.pallas_call(
        functools.partial(_tc_post_body, has_skip, False),
        out_shape=(
            jax.ShapeDtypeStruct((N, H), jnp.float32),
            jax.ShapeDtypeStruct((N, H), jnp.float32),
        ),
    )(*args)


def _tc_final(args):
    return pl.pallas_call(
        functools.partial(_tc_post_body, True, True),
        out_shape=jax.ShapeDtypeStruct((N, 1), jnp.float32),
    )(*args)


def kernel(x, edge_index, batch, params):
    p = params
    wr = p['edge_weight'].reshape(1, 1)
    row, col = edge_index[0], edge_index[1]
    pad0 = jnp.zeros((PADI,), jnp.int32)
    padd = jnp.full((PADI,), DUMMY, jnp.int32)
    srcg = jnp.concatenate([row, col, pad0])
    cidx = jnp.concatenate([row, col, padd])
    dst = jnp.concatenate([col, row, padd])

    cp = _sc_cnt(cidx)
    h, y, dinv = _tc_in(x, p['in_W1'], p['in_b1'], p['in_W2'], p['in_b2'],
                        p['gcn_W0'], cp, wr)

    for i in range(NUM_LAYERS):
        zp = _sc_msg(y, srcg, dst)
        gb = p['gcn_b%d' % i].reshape(1, H)
        ms = p['norm_ms%d' % i].reshape(1, H)
        nw = p['norm_w%d' % i].reshape(1, H)
        nb = p['norm_b%d' % i].reshape(1, H)
        if i < NUM_LAYERS - 1:
            args = [zp, y, h, dinv, wr, gb, ms, nw, nb]
            has_skip = i % RES_FREQ == 0
            if has_skip:
                args += [p['skip_W%d' % i], p['skip_b%d' % i].reshape(1, H)]
            args += [p['gcn_W%d' % (i + 1)]]
            h, y = _tc_mid(has_skip, args)
        else:
            args = [zp, y, h, dinv, wr, gb, ms, nw, nb,
                    p['skip_W%d' % i], p['skip_b%d' % i].reshape(1, H),
                    p['head_W1'], p['head_b1'].reshape(1, H),
                    p['head_W2'], p['head_b2'].reshape(1, H // 2),
                    p['head_W3'], p['head_b3'].reshape(1, 1)]
            out = _tc_final(args)
    return out.reshape(-1)


# speedup vs baseline: 2.8298x; 2.8269x over previous
import functools
import jax
import jax.numpy as jnp
from jax import lax
from jax.experimental import pallas as pl
from jax.experimental.pallas import tpu as pltpu
from jax.experimental.pallas import tpu_sc as plsc

N = 10000
H = 128
NUM_LAYERS = 3
RES_FREQ = 2

NW = 32
CH = 128
E2 = 640000
NCH = 160
EPW = NCH * CH
EP = NW * EPW
IXR = EP // CH + 8
PADI = IXR * CH - E2
ZR = 10240
DUMMY = N
RPT = ZR // 16
ORB = 624
CW = 128

_SC_MESH = plsc.VectorSubcoreMesh(core_axis_name="c", subcore_axis_name="s")


def _zero_rows(ref, nrows, width):
    z = jnp.zeros((16,), jnp.float32)

    def body(i, _):
        for j in range(width // 16):
            ref[i, pl.ds(j * 16, 16)] = z
        return 0

    lax.fori_loop(0, nrows, body, 0)


def _staged_readout(acc, stage, out_hbm, c, s):
    for t in range(5):
        base = s * ORB + t * CH
        pltpu.sync_copy(acc.at[pl.ds(base, CH)], stage)
        pltpu.sync_copy(stage, out_hbm.at[c, pl.ds(base, CH)])


@functools.partial(
    pl.kernel,
    mesh=_SC_MESH,
    out_type=jax.ShapeDtypeStruct((2, N, H), jnp.float32),
    scratch_types=[
        pltpu.VMEM((CH,), jnp.int32),
        pltpu.VMEM((CH,), jnp.int32),
        pltpu.VMEM((CH,), jnp.int32),
        pltpu.VMEM((CH,), jnp.int32),
        pltpu.VMEM((CH, H), jnp.float32),
        pltpu.VMEM((CH, H), jnp.float32),
        pltpu.VMEM_SHARED((ZR, H), jnp.float32),
        pltpu.SemaphoreType.DMA,
        pltpu.SemaphoreType.DMA,
    ],
)
def _sc_msg(y_hbm, src_hbm, dst_hbm, z_hbm, sidx0, didx0, sidx1, didx1,
            rows0, rows1, zsh, sem0, sem1):
    c = lax.axis_index("c")
    s = lax.axis_index("s")
    wid = s * 2 + c
    base = wid * EPW

    _zero_rows(rows0, CH, H)
    for t in range(RPT // CH):
        pltpu.sync_copy(rows0, zsh.at[pl.ds(s * RPT + t * CH, CH)])
    plsc.subcore_barrier()

    def load_idx(off, sidx, didx):
        pltpu.sync_copy(src_hbm.at[pl.ds(off, CH)], sidx)
        pltpu.sync_copy(dst_hbm.at[pl.ds(off, CH)], didx)

    load_idx(base, sidx0, didx0)
    pltpu.async_copy(y_hbm.at[sidx0], rows0, sem0)

    def pair(j, _):
        i = j * 2
        load_idx(base + (i + 1) * CH, sidx1, didx1)
        pltpu.async_copy(y_hbm.at[sidx1], rows1, sem1)
        pltpu.make_async_copy(y_hbm.at[sidx0], rows0, sem0).wait()
        pltpu.sync_copy(rows0, zsh.at[didx0], add=True)
        load_idx(base + (i + 2) * CH, sidx0, didx0)
        pltpu.async_copy(y_hbm.at[sidx0], rows0, sem0)
        pltpu.make_async_copy(y_hbm.at[sidx1], rows1, sem1).wait()
        pltpu.sync_copy(rows1, zsh.at[didx1], add=True)
        return 0

    lax.fori_loop(0, NCH // 2, pair, 0)
    pltpu.make_async_copy(y_hbm.at[sidx0], rows0, sem0).wait()
    plsc.subcore_barrier()
    _staged_readout(zsh, rows0, z_hbm, c, s)


@functools.partial(
    pl.kernel,
    mesh=_SC_MESH,
    out_type=jax.ShapeDtypeStruct((2, N, CW), jnp.float32),
    scratch_types=[
        pltpu.VMEM((CH,), jnp.int32),
        pltpu.VMEM((CH,), jnp.int32),
        pltpu.VMEM((CH, CW), jnp.float32),
        pltpu.VMEM_SHARED((ZR, CW), jnp.float32),
        pltpu.SemaphoreType.DMA,
        pltpu.SemaphoreType.DMA,
    ],
)
def _sc_cnt(idx_hbm, c_hbm, ci0, ci1, obuf, csh, sm0, sm1):
    c = lax.axis_index("c")
    s = lax.axis_index("s")
    wid = s * 2 + c
    base = wid * NCH
    cidx = [ci0, ci1]
    sm = [sm0, sm1]

    _zero_rows(obuf, CH, CW)
    for t in range(RPT // CH):
        pltpu.sync_copy(obuf, csh.at[pl.ds(s * RPT + t * CH, CH)])
    plsc.subcore_barrier()

    one = jnp.ones((16,), jnp.float32)

    def fill(i, _):
        for j in range(CW // 16):
            obuf[i, pl.ds(j * 16, 16)] = one
        return 0

    lax.fori_loop(0, CH, fill, 0)

    def idx_load(i, q):
        pltpu.async_copy(idx_hbm.at[pl.ds((base + i) * CH, CH)], cidx[q],
                         sm[q])

    def idx_wait(q):
        pltpu.make_async_copy(idx_hbm.at[pl.ds(base * CH, CH)], cidx[q],
                              sm[q]).wait()

    idx_load(0, 0)
    idx_load(1, 1)

    def body(g, _):
        i0 = g * 2
        for q in range(2):
            idx_wait(q)
            pltpu.sync_copy(obuf, csh.at[cidx[q]], add=True)
            idx_load(i0 + q + 2, q)
        return 0

    lax.fori_loop(0, NCH // 2, body, 0)
    idx_wait(0)
    idx_wait(1)
    plsc.subcore_barrier()
    _staged_readout(csh, obuf, c_hbm, c, s)


def _relu(v):
    return jnp.maximum(v, 0.0)


def _mm(a, b):
    return jnp.dot(a, b, preferred_element_type=jnp.float32)


def _tc_in_body(x, w1, b1, w2, b2, wg, cp, wr, oh, oy, od):
    h = _mm(_relu(_mm(x[...], w1[...]) + b1[...]), w2[...]) + b2[...]
    cnt = (cp[0] + cp[1])[:, 0:1]
    dinv = lax.rsqrt(wr[0, 0] * cnt + 2.0)
    oh[...] = h
    od[...] = dinv
    oy[...] = dinv * _mm(h, wg[...])


def _tc_in(x, w1, b1, w2, b2, wg, cp, wr):
    return pl.pallas_call(
        _tc_in_body,
        out_shape=(
            jax.ShapeDtypeStruct((N, H), jnp.float32),
            jax.ShapeDtypeStruct((N, H), jnp.float32),
            jax.ShapeDtypeStruct((N, 1), jnp.float32),
        ),
    )(x, w1, b1.reshape(1, H), w2, b2.reshape(1, H), wg, cp, wr)


def _tc_post_body(has_skip, is_final, *refs):
    if is_final:
        (zp, y, h, dinv, wr, gb, ms, nw, nb, skw, skb,
         hw1, hb1, hw2, hb2, hw3, hb3, out) = refs
    elif has_skip:
        (zp, y, h, dinv, wr, gb, ms, nw, nb, skw, skb, wgn, oh, oy) = refs
    else:
        (zp, y, h, dinv, wr, gb, ms, nw, nb, wgn, oh, oy) = refs
    di = dinv[...]
    g = di * (wr[0, 0] * (zp[0] + zp[1]) + 2.0 * y[...]) + gb[...]
    mean = jnp.mean(g, axis=0, keepdims=True)
    o = g - ms[...] * mean
    var = jnp.mean(o * o, axis=0, keepdims=True)
    g = nw[...] * o * lax.rsqrt(var + 1e-5) + nb[...]
    g = jnp.where(g > 0, g, jnp.exp(g) - 1.0)
    if has_skip:
        g = g + _mm(h[...], skw[...]) + skb[...]
    if is_final:
        q = _relu(_mm(g, hw1[...]) + hb1[...])
        q = _relu(_mm(q, hw2[...]) + hb2[...])
        q = _mm(q, hw3[...]) + hb3[...]
        out[...] = 1.0 / (1.0 + jnp.exp(-q))
    else:
        oh[...] = g
        oy[...] = di * _mm(g, wgn[...])


def _tc_mid(has_skip, args):
    return pl.pallas_call(
        functools.partial(_tc_post_body, has_skip, False),
        out_shape=(
            jax.ShapeDtypeStruct((N, H), jnp.float32),
            jax.ShapeDtypeStruct((N, H), jnp.float32),
        ),
    )(*args)


def _tc_final(args):
    return pl.pallas_call(
        functools.partial(_tc_post_body, True, True),
        out_shape=jax.ShapeDtypeStruct((N, 1), jnp.float32),
    )(*args)


def kernel(x, edge_index, batch, params):
    p = params
    wr = p['edge_weight'].reshape(1, 1)
    row, col = edge_index[0], edge_index[1]
    spread = jnp.arange(PADI, dtype=jnp.int32) % (ZR - N)
    pad0 = spread
    padd = DUMMY + spread
    srcg = jnp.concatenate([row, col, pad0])
    cidx = jnp.concatenate([row, col, padd])
    dst = jnp.concatenate([col, row, padd])

    cp = _sc_cnt(cidx)
    h, y, dinv = _tc_in(x, p['in_W1'], p['in_b1'], p['in_W2'], p['in_b2'],
                        p['gcn_W0'], cp, wr)

    for i in range(NUM_LAYERS):
        zp = _sc_msg(y, srcg, dst)
        gb = p['gcn_b%d' % i].reshape(1, H)
        ms = p['norm_ms%d' % i].reshape(1, H)
        nw = p['norm_w%d' % i].reshape(1, H)
        nb = p['norm_b%d' % i].reshape(1, H)
        if i < NUM_LAYERS - 1:
            args = [zp, y, h, dinv, wr, gb, ms, nw, nb]
            has_skip = i % RES_FREQ == 0
            if has_skip:
                args += [p['skip_W%d' % i], p['skip_b%d' % i].reshape(1, H)]
            args += [p['gcn_W%d' % (i + 1)]]
            h, y = _tc_mid(has_skip, args)
        else:
            args = [zp, y, h, dinv, wr, gb, ms, nw, nb,
                    p['skip_W%d' % i], p['skip_b%d' % i].reshape(1, H),
                    p['head_W1'], p['head_b1'].reshape(1, H),
                    p['head_W2'], p['head_b2'].reshape(1, H // 2),
                    p['head_W3'], p['head_b3'].reshape(1, 1)]
            out = _tc_final(args)
    return out.reshape(-1)


# speedup vs baseline: 2.8301x; 1.0001x over previous
import functools
import jax
import jax.numpy as jnp
from jax import lax
from jax.experimental import pallas as pl
from jax.experimental.pallas import tpu as pltpu
from jax.experimental.pallas import tpu_sc as plsc

N = 10000
H = 128
NUM_LAYERS = 3
RES_FREQ = 2

NW = 32
CH = 128
E2 = 640000
NCH = 160
EPW = NCH * CH
EP = NW * EPW
IXR = EP // CH + 8
PADI = IXR * CH - E2
ZR = 10112
DUMMY = N
RPT = ZR // 16
ORB = 624
CW = 128

_SC_MESH = plsc.VectorSubcoreMesh(core_axis_name="c", subcore_axis_name="s")


def _zero_acc(zbuf, acc, s):
    for t in range(RPT // CH):
        pltpu.sync_copy(zbuf, acc.at[pl.ds(s * RPT + t * CH, CH)])
    rem = RPT % CH
    if rem:
        pltpu.sync_copy(zbuf.at[pl.ds(0, rem)],
                        acc.at[pl.ds(s * RPT + (RPT // CH) * CH, rem)])


def _zero_rows(ref, nrows, width):
    z = jnp.zeros((16,), jnp.float32)

    def body(i, _):
        for j in range(width // 16):
            ref[i, pl.ds(j * 16, 16)] = z
        return 0

    lax.fori_loop(0, nrows, body, 0)


def _staged_readout(acc, stage, out_hbm, c, s):
    for t in range(5):
        base = s * ORB + t * CH
        pltpu.sync_copy(acc.at[pl.ds(base, CH)], stage)
        pltpu.sync_copy(stage, out_hbm.at[c, pl.ds(base, CH)])


@functools.partial(
    pl.kernel,
    mesh=_SC_MESH,
    out_type=jax.ShapeDtypeStruct((2, N, H), jnp.float32),
    scratch_types=[
        pltpu.VMEM((CH,), jnp.int32),
        pltpu.VMEM((CH,), jnp.int32),
        pltpu.VMEM((CH,), jnp.int32),
        pltpu.VMEM((CH,), jnp.int32),
        pltpu.VMEM((CH,), jnp.int32),
        pltpu.VMEM((CH,), jnp.int32),
        pltpu.VMEM((CH, H), jnp.float32),
        pltpu.VMEM((CH, H), jnp.float32),
        pltpu.VMEM((CH, H), jnp.float32),
        pltpu.VMEM_SHARED((ZR, H), jnp.float32),
        pltpu.SemaphoreType.DMA,
        pltpu.SemaphoreType.DMA,
        pltpu.SemaphoreType.DMA,
    ],
)
def _sc_msg(y_hbm, src_hbm, dst_hbm, z_hbm, si0, di0, si1, di1, si2, di2,
            rows0, rows1, rows2, zsh, sem0, sem1, sem2):
    c = lax.axis_index("c")
    s = lax.axis_index("s")
    wid = s * 2 + c
    base = wid * EPW
    sidx = [si0, si1, si2]
    didx = [di0, di1, di2]
    rows = [rows0, rows1, rows2]
    sems = [sem0, sem1, sem2]

    _zero_rows(rows0, CH, H)
    _zero_acc(rows0, zsh, s)
    plsc.subcore_barrier()

    def load_idx(i, q):
        off = base + i * CH
        pltpu.sync_copy(src_hbm.at[pl.ds(off, CH)], sidx[q])
        pltpu.sync_copy(dst_hbm.at[pl.ds(off, CH)], didx[q])

    for q in range(3):
        load_idx(q, q)
        pltpu.async_copy(y_hbm.at[sidx[q]], rows[q], sems[q])

    def triple(j, _):
        i0 = j * 3
        for q in range(3):
            i = i0 + q
            pltpu.make_async_copy(y_hbm.at[sidx[q]], rows[q], sems[q]).wait()
            pltpu.sync_copy(rows[q], zsh.at[didx[q]], add=True)
            load_idx(i + 3, q)
            pltpu.async_copy(y_hbm.at[sidx[q]], rows[q], sems[q])
        return 0

    lax.fori_loop(0, (NCH - 1) // 3, triple, 0)
    pltpu.make_async_copy(y_hbm.at[si0], rows0, sem0).wait()
    pltpu.sync_copy(rows0, zsh.at[di0], add=True)
    pltpu.make_async_copy(y_hbm.at[si1], rows1, sem1).wait()
    pltpu.make_async_copy(y_hbm.at[si2], rows2, sem2).wait()
    plsc.subcore_barrier()
    _staged_readout(zsh, rows0, z_hbm, c, s)


@functools.partial(
    pl.kernel,
    mesh=_SC_MESH,
    out_type=jax.ShapeDtypeStruct((2, N, CW), jnp.float32),
    scratch_types=[
        pltpu.VMEM((CH,), jnp.int32),
        pltpu.VMEM((CH,), jnp.int32),
        pltpu.VMEM((CH, CW), jnp.float32),
        pltpu.VMEM_SHARED((ZR, CW), jnp.float32),
        pltpu.SemaphoreType.DMA,
        pltpu.SemaphoreType.DMA,
    ],
)
def _sc_cnt(idx_hbm, c_hbm, ci0, ci1, obuf, csh, sm0, sm1):
    c = lax.axis_index("c")
    s = lax.axis_index("s")
    wid = s * 2 + c
    base = wid * NCH
    cidx = [ci0, ci1]
    sm = [sm0, sm1]

    _zero_rows(obuf, CH, CW)
    _zero_acc(obuf, csh, s)
    plsc.subcore_barrier()

    one = jnp.ones((16,), jnp.float32)

    def fill(i, _):
        for j in range(CW // 16):
            obuf[i, pl.ds(j * 16, 16)] = one
        return 0

    lax.fori_loop(0, CH, fill, 0)

    def idx_load(i, q):
        pltpu.async_copy(idx_hbm.at[pl.ds((base + i) * CH, CH)], cidx[q],
                         sm[q])

    def idx_wait(q):
        pltpu.make_async_copy(idx_hbm.at[pl.ds(base * CH, CH)], cidx[q],
                              sm[q]).wait()

    idx_load(0, 0)
    idx_load(1, 1)

    def body(g, _):
        i0 = g * 2
        for q in range(2):
            idx_wait(q)
            pltpu.sync_copy(obuf, csh.at[cidx[q]], add=True)
            idx_load(i0 + q + 2, q)
        return 0

    lax.fori_loop(0, NCH // 2, body, 0)
    idx_wait(0)
    idx_wait(1)
    plsc.subcore_barrier()
    _staged_readout(csh, obuf, c_hbm, c, s)


def _relu(v):
    return jnp.maximum(v, 0.0)


def _mm(a, b):
    return jnp.dot(a, b, preferred_element_type=jnp.float32)


def _tc_in_body(x, w1, b1, w2, b2, wg, cp, wr, oh, oy, od):
    h = _mm(_relu(_mm(x[...], w1[...]) + b1[...]), w2[...]) + b2[...]
    cnt = (cp[0] + cp[1])[:, 0:1]
    dinv = lax.rsqrt(wr[0, 0] * cnt + 2.0)
    oh[...] = h
    od[...] = dinv
    oy[...] = dinv * _mm(h, wg[...])


def _tc_in(x, w1, b1, w2, b2, wg, cp, wr):
    return pl.pallas_call(
        _tc_in_body,
        out_shape=(
            jax.ShapeDtypeStruct((N, H), jnp.float32),
            jax.ShapeDtypeStruct((N, H), jnp.float32),
            jax.ShapeDtypeStruct((N, 1), jnp.float32),
        ),
    )(x, w1, b1.reshape(1, H), w2, b2.reshape(1, H), wg, cp, wr)


def _tc_post_body(has_skip, is_final, *refs):
    if is_final:
        (zp, y, h, dinv, wr, gb, ms, nw, nb, skw, skb,
         hw1, hb1, hw2, hb2, hw3, hb3, out) = refs
    elif has_skip:
        (zp, y, h, dinv, wr, gb, ms, nw, nb, skw, skb, wgn, oh, oy) = refs
    else:
        (zp, y, h, dinv, wr, gb, ms, nw, nb, wgn, oh, oy) = refs
    di = dinv[...]
    g = di * (wr[0, 0] * (zp[0] + zp[1]) + 2.0 * y[...]) + gb[...]
    mean = jnp.mean(g, axis=0, keepdims=True)
    o = g - ms[...] * mean
    var = jnp.mean(o * o, axis=0, keepdims=True)
    g = nw[...] * o * lax.rsqrt(var + 1e-5) + nb[...]
    g = jnp.where(g > 0, g, jnp.exp(g) - 1.0)
    if has_skip:
        g = g + _mm(h[...], skw[...]) + skb[...]
    if is_final:
        q = _relu(_mm(g, hw1[...]) + hb1[...])
        q = _relu(_mm(q, hw2[...]) + hb2[...])
        q = _mm(q, hw3[...]) + hb3[...]
        out[...] = 1.0 / (1.0 + jnp.exp(-q))
    else:
        oh[...] = g
        oy[...] = di * _mm(g, wgn[...])


def _tc_mid(has_skip, args):
    return pl.pallas_call(
        functools.partial(_tc_post_body, has_skip, False),
        out_shape=(
            jax.ShapeDtypeStruct((N, H), jnp.float32),
            jax.ShapeDtypeStruct((N, H), jnp.float32),
        ),
    )(*args)


def _tc_final(args):
    return pl.pallas_call(
        functools.partial(_tc_post_body, True, True),
        out_shape=jax.ShapeDtypeStruct((N, 1), jnp.float32),
    )(*args)


def kernel(x, edge_index, batch, params):
    p = params
    wr = p['edge_weight'].reshape(1, 1)
    row, col = edge_index[0], edge_index[1]
    spread = jnp.arange(PADI, dtype=jnp.int32) % (ZR - N)
    pad0 = spread
    padd = DUMMY + spread
    srcg = jnp.concatenate([row, col, pad0])
    cidx = jnp.concatenate([row, col, padd])
    dst = jnp.concatenate([col, row, padd])

    cp = _sc_cnt(cidx)
    h, y, dinv = _tc_in(x, p['in_W1'], p['in_b1'], p['in_W2'], p['in_b2'],
                        p['gcn_W0'], cp, wr)

    for i in range(NUM_LAYERS):
        zp = _sc_msg(y, srcg, dst)
        gb = p['gcn_b%d' % i].reshape(1, H)
        ms = p['norm_ms%d' % i].reshape(1, H)
        nw = p['norm_w%d' % i].reshape(1, H)
        nb = p['norm_b%d' % i].reshape(1, H)
        if i < NUM_LAYERS - 1:
            args = [zp, y, h, dinv, wr, gb, ms, nw, nb]
            has_skip = i % RES_FREQ == 0
            if has_skip:
                args += [p['skip_W%d' % i], p['skip_b%d' % i].reshape(1, H)]
            args += [p['gcn_W%d' % (i + 1)]]
            h, y = _tc_mid(has_skip, args)
        else:
            args = [zp, y, h, dinv, wr, gb, ms, nw, nb,
                    p['skip_W%d' % i], p['skip_b%d' % i].reshape(1, H),
                    p['head_W1'], p['head_b1'].reshape(1, H),
                    p['head_W2'], p['head_b2'].reshape(1, H // 2),
                    p['head_W3'], p['head_b3'].reshape(1, 1)]
            out = _tc_final(args)
    return out.reshape(-1)
